# baseline (speedup 1.0000x reference)
import functools

import jax
import jax.numpy as jnp
from jax import lax
from jax.experimental import pallas as pl
from jax.experimental.pallas import tpu as pltpu
from jax.experimental.pallas import tpu_sc as plsc

N_NODES = 10000
NODE_SIZE = 128
NUM_GRAPHS = 128
GLOBAL_SIZE = 64
NC = 2
NS = 16
L = 16
C = 192
SC_ROWS = NC * NS * C
SEG_ROWS = 136
ZROWS = 16
CHUNKS = (128, 64)
HBLK = 512

_SELU_SCALE = 1.0507009873554805
_SELU_ALPHA = 1.6732632423543772


def _sc_segsum(x, batch):
    mesh = plsc.VectorSubcoreMesh(core_axis_name="c", subcore_axis_name="s")

    @functools.partial(
        pl.kernel,
        mesh=mesh,
        out_type=jax.ShapeDtypeStruct((NC * SEG_ROWS, NODE_SIZE), jnp.float32),
        scratch_types=[
            pltpu.VMEM((C, NODE_SIZE), jnp.float32),
            pltpu.VMEM((C,), jnp.int32),
            pltpu.VMEM((len(CHUNKS), 128), jnp.int32),
            pltpu.VMEM((ZROWS, NODE_SIZE), jnp.float32),
            pltpu.VMEM_SHARED((SEG_ROWS, NODE_SIZE), jnp.float32),
            pltpu.SemaphoreType.DMA,
            pltpu.SemaphoreType.DMA,
            pltpu.SemaphoreType.DMA,
        ],
    )
    def seg_kernel(x_hbm, b_hbm, acc_out, xv, bv, idx2, zbuf, acc_sh,
                   semx, semb, sems):
        cid = lax.axis_index("c")
        sid = lax.axis_index("s")
        wid = sid * NC + cid
        base = wid * C

        hx = pltpu.async_copy(x_hbm.at[pl.ds(base, C), :], xv, semx)
        hb = pltpu.async_copy(b_hbm.at[pl.ds(base, C)], bv, semb)

        zero = jnp.zeros((L,), jnp.float32)
        for i in range(ZROWS):
            for j in range(NODE_SIZE // L):
                zbuf[i, pl.ds(j * L, L)] = zero
        zbase = jnp.minimum(sid * ZROWS, SEG_ROWS - ZROWS)
        pltpu.sync_copy(zbuf, acc_sh.at[pl.ds(zbase, ZROWS)])

        hb.wait()
        bounds = []
        off = 0
        for csz in CHUNKS:
            bounds.append(off)
            off += csz
        for k in range(C // L):
            p = k * L
            ci = sum(1 for lo in bounds if lo <= p) - 1
            idx2[ci, pl.ds(p - bounds[ci], L)] = bv[pl.ds(p, L)]

        hx.wait()
        plsc.subcore_barrier()

        handles = []
        off = 0
        for ci, csz in enumerate(CHUNKS):
            handles.append(pltpu.async_copy(
                xv.at[pl.ds(off, csz), :],
                acc_sh.at[idx2.at[ci, pl.ds(0, csz)]], sems, add=True))
            off += csz
        for h in handles:
            h.wait()

        plsc.subcore_barrier()

        @pl.when(sid == 0)
        def _():
            pltpu.sync_copy(acc_sh, acc_out.at[pl.ds(cid * SEG_ROWS, SEG_ROWS)])

    return seg_kernel(x, batch)


def _tc_hist_tail(b_ref, x_ref, cnt_ref, tsum_ref, xbuf, sem0, sem1):
    sems = (sem0, sem1)
    blocks = []
    for k in range(0, N_NODES, HBLK):
        blocks.append((k, min(HBLK, N_NODES - k)))
    tail = [(o, s) for (o, s) in blocks if o >= SC_ROWS]

    handles = {}
    for i in range(min(2, len(tail))):
        o, s = tail[i]
        handles[i] = pltpu.make_async_copy(
            x_ref.at[pl.ds(o, s), :], xbuf.at[i % 2, pl.ds(0, s), :],
            sems[i % 2])
        handles[i].start()

    gid = lax.broadcasted_iota(jnp.int32, (NUM_GRAPHS, 1), 0)
    cnt = jnp.zeros((NUM_GRAPHS, 1), jnp.float32)
    tsum = jnp.zeros((NUM_GRAPHS, NODE_SIZE), jnp.float32)
    ti = 0
    for (o, s) in blocks:
        eq = (b_ref[:, o:o + s] == gid).astype(jnp.float32)
        cnt = cnt + jnp.sum(eq, axis=1, keepdims=True)
        if o >= SC_ROWS:
            handles[ti].wait()
            if ti + 2 < len(tail):
                o2, s2 = tail[ti + 2]
                handles[ti + 2] = pltpu.make_async_copy(
                    x_ref.at[pl.ds(o2, s2), :],
                    xbuf.at[ti % 2, pl.ds(0, s2), :], sems[ti % 2])
                handles[ti + 2].start()
            xblk = xbuf[ti % 2, pl.ds(0, s), :]
            tsum = tsum + jnp.dot(eq, xblk,
                                  preferred_element_type=jnp.float32)
            ti += 1
    cnt_ref[:] = jnp.maximum(cnt, 1.0)
    tsum_ref[:] = tsum


def _tc_mlp(acc_ref, ts_ref, cnt_ref, u_ref, w1_ref, b1_ref, w2_ref,
            b2_ref, out_ref):
    s = (acc_ref[:NUM_GRAPHS, :]
         + acc_ref[SEG_ROWS:SEG_ROWS + NUM_GRAPHS, :]
         + ts_ref[:])
    mean = s / cnt_ref[:]
    h = (jnp.dot(u_ref[:], w1_ref[:GLOBAL_SIZE, :],
                 preferred_element_type=jnp.float32)
         + jnp.dot(mean, w1_ref[GLOBAL_SIZE:, :],
                   preferred_element_type=jnp.float32)
         + b1_ref[:])
    h = _SELU_SCALE * jnp.where(h > 0, h, _SELU_ALPHA * (jnp.exp(h) - 1.0))
    out_t = lax.dot_general(w2_ref[:], h, (((0,), (1,)), ((), ())),
                            preferred_element_type=jnp.float32)
    out_ref[:] = out_t + b2_ref[:]


def kernel(x, edge_index, edge_attr, u, batch, W1, b1, W2, b2):
    acc = _sc_segsum(x, batch)
    cnt, tsum = pl.pallas_call(
        _tc_hist_tail,
        in_specs=[
            pl.BlockSpec(memory_space=pltpu.VMEM),
            pl.BlockSpec(memory_space=pl.ANY),
        ],
        out_shape=(
            jax.ShapeDtypeStruct((NUM_GRAPHS, 1), jnp.float32),
            jax.ShapeDtypeStruct((NUM_GRAPHS, NODE_SIZE), jnp.float32),
        ),
        scratch_shapes=[
            pltpu.VMEM((2, HBLK, NODE_SIZE), jnp.float32),
            pltpu.SemaphoreType.DMA,
            pltpu.SemaphoreType.DMA,
        ],
    )(batch.reshape(1, N_NODES), x)
    out_t = pl.pallas_call(
        _tc_mlp,
        out_shape=jax.ShapeDtypeStruct((W2.shape[1], NUM_GRAPHS), jnp.float32),
    )(acc, tsum, cnt, u, W1, b1.reshape(1, -1), W2, b2.reshape(-1, 1))
    return out_t.T


# speedup vs baseline: 1.0674x; 1.0674x over previous
import functools

import jax
import jax.numpy as jnp
from jax import lax
from jax.experimental import pallas as pl
from jax.experimental.pallas import tpu as pltpu
from jax.experimental.pallas import tpu_sc as plsc

N_NODES = 10000
NODE_SIZE = 128
NUM_GRAPHS = 128
GLOBAL_SIZE = 64
NC = 2
NS = 16
L = 16
C = 208
SC_ROWS = NC * NS * C
SEG_ROWS = 136
ZROWS = 16
CHUNKS = (128, 80)
HBLK = 512

_SELU_SCALE = 1.0507009873554805
_SELU_ALPHA = 1.6732632423543772


def _sc_segsum(x, batch):
    mesh = plsc.VectorSubcoreMesh(core_axis_name="c", subcore_axis_name="s")

    @functools.partial(
        pl.kernel,
        mesh=mesh,
        out_type=jax.ShapeDtypeStruct((NC * SEG_ROWS, NODE_SIZE), jnp.float32),
        scratch_types=[
            pltpu.VMEM((C, NODE_SIZE), jnp.float32),
            pltpu.VMEM((C,), jnp.int32),
            pltpu.VMEM((len(CHUNKS), 128), jnp.int32),
            pltpu.VMEM((ZROWS, NODE_SIZE), jnp.float32),
            pltpu.VMEM_SHARED((SEG_ROWS, NODE_SIZE), jnp.float32),
            pltpu.SemaphoreType.DMA,
            pltpu.SemaphoreType.DMA,
            pltpu.SemaphoreType.DMA,
        ],
    )
    def seg_kernel(x_hbm, b_hbm, acc_out, xv, bv, idx2, zbuf, acc_sh,
                   semx, semb, sems):
        cid = lax.axis_index("c")
        sid = lax.axis_index("s")
        wid = sid * NC + cid
        base = wid * C

        hx = pltpu.async_copy(x_hbm.at[pl.ds(base, C), :], xv, semx)
        hb = pltpu.async_copy(b_hbm.at[pl.ds(base, C)], bv, semb)

        zero = jnp.zeros((L,), jnp.float32)
        for i in range(ZROWS):
            for j in range(NODE_SIZE // L):
                zbuf[i, pl.ds(j * L, L)] = zero
        zbase = jnp.minimum(sid * ZROWS, SEG_ROWS - ZROWS)
        pltpu.sync_copy(zbuf, acc_sh.at[pl.ds(zbase, ZROWS)])

        hb.wait()
        bounds = []
        off = 0
        for csz in CHUNKS:
            bounds.append(off)
            off += csz
        for k in range(C // L):
            p = k * L
            ci = sum(1 for lo in bounds if lo <= p) - 1
            idx2[ci, pl.ds(p - bounds[ci], L)] = bv[pl.ds(p, L)]

        hx.wait()
        plsc.subcore_barrier()

        handles = []
        off = 0
        for ci, csz in enumerate(CHUNKS):
            handles.append(pltpu.async_copy(
                xv.at[pl.ds(off, csz), :],
                acc_sh.at[idx2.at[ci, pl.ds(0, csz)]], sems, add=True))
            off += csz
        for h in handles:
            h.wait()

        plsc.subcore_barrier()

        @pl.when(sid == 0)
        def _():
            pltpu.sync_copy(acc_sh, acc_out.at[pl.ds(cid * SEG_ROWS, SEG_ROWS)])

    return seg_kernel(x, batch)


def _tc_hist_tail(b_ref, x_ref, cnt_ref, tsum_ref, xbuf, sem0, sem1):
    sems = (sem0, sem1)
    blocks = []
    for k in range(0, N_NODES, HBLK):
        blocks.append((k, min(HBLK, N_NODES - k)))
    tail = [(o, s) for (o, s) in blocks if o >= SC_ROWS]

    handles = {}
    for i in range(min(2, len(tail))):
        o, s = tail[i]
        handles[i] = pltpu.make_async_copy(
            x_ref.at[pl.ds(o, s), :], xbuf.at[i % 2, pl.ds(0, s), :],
            sems[i % 2])
        handles[i].start()

    gid = lax.broadcasted_iota(jnp.int32, (NUM_GRAPHS, 1), 0)
    cnt = jnp.zeros((NUM_GRAPHS, 1), jnp.float32)
    tsum = jnp.zeros((NUM_GRAPHS, NODE_SIZE), jnp.float32)
    ti = 0
    for (o, s) in blocks:
        eq = (b_ref[:, o:o + s] == gid).astype(jnp.float32)
        cnt = cnt + jnp.sum(eq, axis=1, keepdims=True)
        if o >= SC_ROWS:
            handles[ti].wait()
            if ti + 2 < len(tail):
                o2, s2 = tail[ti + 2]
                handles[ti + 2] = pltpu.make_async_copy(
                    x_ref.at[pl.ds(o2, s2), :],
                    xbuf.at[ti % 2, pl.ds(0, s2), :], sems[ti % 2])
                handles[ti + 2].start()
            xblk = xbuf[ti % 2, pl.ds(0, s), :]
            tsum = tsum + jnp.dot(eq, xblk,
                                  preferred_element_type=jnp.float32)
            ti += 1
    cnt_ref[:] = jnp.maximum(cnt, 1.0)
    tsum_ref[:] = tsum


def _tc_mlp(acc_ref, ts_ref, cnt_ref, ut_ref, w1t_ref, b1_ref, w2_ref,
            b2_ref, out_ref):
    s = (acc_ref[:NUM_GRAPHS, :]
         + acc_ref[SEG_ROWS:SEG_ROWS + NUM_GRAPHS, :]
         + ts_ref[:])
    mean = s / cnt_ref[:]
    h = (lax.dot_general(ut_ref[:], w1t_ref[:, :GLOBAL_SIZE],
                         (((0,), (1,)), ((), ())),
                         preferred_element_type=jnp.float32)
         + lax.dot_general(mean, w1t_ref[:, GLOBAL_SIZE:],
                           (((1,), (1,)), ((), ())),
                           preferred_element_type=jnp.float32)
         + b1_ref[:])
    h = _SELU_SCALE * jnp.where(h > 0, h, _SELU_ALPHA * (jnp.exp(h) - 1.0))
    out_t = lax.dot_general(w2_ref[:], h, (((0,), (1,)), ((), ())),
                            preferred_element_type=jnp.float32)
    out_ref[:] = out_t + b2_ref[:]


def kernel(x, edge_index, edge_attr, u, batch, W1, b1, W2, b2):
    acc = _sc_segsum(x, batch)
    cnt, tsum = pl.pallas_call(
        _tc_hist_tail,
        in_specs=[
            pl.BlockSpec(memory_space=pltpu.VMEM),
            pl.BlockSpec(memory_space=pl.ANY),
        ],
        out_shape=(
            jax.ShapeDtypeStruct((NUM_GRAPHS, 1), jnp.float32),
            jax.ShapeDtypeStruct((NUM_GRAPHS, NODE_SIZE), jnp.float32),
        ),
        scratch_shapes=[
            pltpu.VMEM((2, HBLK, NODE_SIZE), jnp.float32),
            pltpu.SemaphoreType.DMA,
            pltpu.SemaphoreType.DMA,
        ],
    )(batch.reshape(1, N_NODES), x)
    out_t = pl.pallas_call(
        _tc_mlp,
        out_shape=jax.ShapeDtypeStruct((W2.shape[1], NUM_GRAPHS), jnp.float32),
    )(acc, tsum, cnt, u.T, W1.T, b1.reshape(1, -1), W2, b2.reshape(-1, 1))
    return out_t.T


# speedup vs baseline: 1.0776x; 1.0096x over previous
import functools

import jax
import jax.numpy as jnp
from jax import lax
from jax.experimental import pallas as pl
from jax.experimental.pallas import tpu as pltpu
from jax.experimental.pallas import tpu_sc as plsc

N_NODES = 10000
NODE_SIZE = 128
NUM_GRAPHS = 128
GLOBAL_SIZE = 64
NC = 2
NS = 16
L = 16
C = 240
SC_ROWS = NC * NS * C
SEG_ROWS = 136
ZROWS = 16
CHUNKS = (128, 112)
HBLK = 512

_SELU_SCALE = 1.0507009873554805
_SELU_ALPHA = 1.6732632423543772


def _sc_segsum(x, batch):
    mesh = plsc.VectorSubcoreMesh(core_axis_name="c", subcore_axis_name="s")

    @functools.partial(
        pl.kernel,
        mesh=mesh,
        out_type=jax.ShapeDtypeStruct((NC * SEG_ROWS, NODE_SIZE), jnp.float32),
        scratch_types=[
            pltpu.VMEM((C, NODE_SIZE), jnp.float32),
            pltpu.VMEM((C,), jnp.int32),
            pltpu.VMEM((len(CHUNKS), 128), jnp.int32),
            pltpu.VMEM((ZROWS, NODE_SIZE), jnp.float32),
            pltpu.VMEM_SHARED((SEG_ROWS, NODE_SIZE), jnp.float32),
            pltpu.SemaphoreType.DMA,
            pltpu.SemaphoreType.DMA,
            pltpu.SemaphoreType.DMA,
        ],
    )
    def seg_kernel(x_hbm, b_hbm, acc_out, xv, bv, idx2, zbuf, acc_sh,
                   semx, semb, sems):
        cid = lax.axis_index("c")
        sid = lax.axis_index("s")
        wid = sid * NC + cid
        base = wid * C

        hx = pltpu.async_copy(x_hbm.at[pl.ds(base, C), :], xv, semx)
        hb = pltpu.async_copy(b_hbm.at[pl.ds(base, C)], bv, semb)

        zero = jnp.zeros((L,), jnp.float32)
        for i in range(ZROWS):
            for j in range(NODE_SIZE // L):
                zbuf[i, pl.ds(j * L, L)] = zero
        zbase = jnp.minimum(sid * ZROWS, SEG_ROWS - ZROWS)
        pltpu.sync_copy(zbuf, acc_sh.at[pl.ds(zbase, ZROWS)])

        hb.wait()
        bounds = []
        off = 0
        for csz in CHUNKS:
            bounds.append(off)
            off += csz
        for k in range(C // L):
            p = k * L
            ci = sum(1 for lo in bounds if lo <= p) - 1
            idx2[ci, pl.ds(p - bounds[ci], L)] = bv[pl.ds(p, L)]

        hx.wait()
        plsc.subcore_barrier()

        handles = []
        off = 0
        for ci, csz in enumerate(CHUNKS):
            handles.append(pltpu.async_copy(
                xv.at[pl.ds(off, csz), :],
                acc_sh.at[idx2.at[ci, pl.ds(0, csz)]], sems, add=True))
            off += csz
        for h in handles:
            h.wait()

        plsc.subcore_barrier()

        @pl.when(sid == 0)
        def _():
            pltpu.sync_copy(acc_sh, acc_out.at[pl.ds(cid * SEG_ROWS, SEG_ROWS)])

    return seg_kernel(x, batch)


def _tc_hist_tail(b_ref, x_ref, cnt_ref, tsum_ref, xbuf, sem0, sem1):
    sems = (sem0, sem1)
    blocks = []
    for k in range(0, N_NODES, HBLK):
        blocks.append((k, min(HBLK, N_NODES - k)))
    tail = [(o, s) for (o, s) in blocks if o >= SC_ROWS]

    handles = {}
    for i in range(min(2, len(tail))):
        o, s = tail[i]
        handles[i] = pltpu.make_async_copy(
            x_ref.at[pl.ds(o, s), :], xbuf.at[i % 2, pl.ds(0, s), :],
            sems[i % 2])
        handles[i].start()

    gid = lax.broadcasted_iota(jnp.int32, (NUM_GRAPHS, 1), 0)
    cnt = jnp.zeros((NUM_GRAPHS, 1), jnp.float32)
    tsum = jnp.zeros((NUM_GRAPHS, NODE_SIZE), jnp.float32)
    ti = 0
    for (o, s) in blocks:
        eq = (b_ref[:, o:o + s] == gid).astype(jnp.float32)
        ones_col = jnp.ones((s, 1), jnp.float32)
        cnt = cnt + jnp.dot(eq, ones_col,
                            preferred_element_type=jnp.float32)
        if o >= SC_ROWS:
            handles[ti].wait()
            if ti + 2 < len(tail):
                o2, s2 = tail[ti + 2]
                handles[ti + 2] = pltpu.make_async_copy(
                    x_ref.at[pl.ds(o2, s2), :],
                    xbuf.at[ti % 2, pl.ds(0, s2), :], sems[ti % 2])
                handles[ti + 2].start()
            xblk = xbuf[ti % 2, pl.ds(0, s), :]
            tsum = tsum + jnp.dot(eq, xblk,
                                  preferred_element_type=jnp.float32)
            ti += 1
    cnt_ref[:] = jnp.maximum(cnt, 1.0)
    tsum_ref[:] = tsum


def _tc_mlp(acc_ref, ts_ref, cnt_ref, ut_ref, w1t_ref, b1_ref, w2_ref,
            b2_ref, out_ref):
    s = (acc_ref[:NUM_GRAPHS, :]
         + acc_ref[SEG_ROWS:SEG_ROWS + NUM_GRAPHS, :]
         + ts_ref[:])
    mean = s / cnt_ref[:]
    h = (lax.dot_general(ut_ref[:], w1t_ref[:, :GLOBAL_SIZE],
                         (((0,), (1,)), ((), ())),
                         preferred_element_type=jnp.float32)
         + lax.dot_general(mean, w1t_ref[:, GLOBAL_SIZE:],
                           (((1,), (1,)), ((), ())),
                           preferred_element_type=jnp.float32)
         + b1_ref[:])
    h = _SELU_SCALE * jnp.where(h > 0, h, _SELU_ALPHA * (jnp.exp(h) - 1.0))
    out_t = lax.dot_general(w2_ref[:], h, (((0,), (1,)), ((), ())),
                            preferred_element_type=jnp.float32)
    out_ref[:] = out_t + b2_ref[:]


def kernel(x, edge_index, edge_attr, u, batch, W1, b1, W2, b2):
    acc = _sc_segsum(x, batch)
    cnt, tsum = pl.pallas_call(
        _tc_hist_tail,
        in_specs=[
            pl.BlockSpec(memory_space=pltpu.VMEM),
            pl.BlockSpec(memory_space=pl.ANY),
        ],
        out_shape=(
            jax.ShapeDtypeStruct((NUM_GRAPHS, 1), jnp.float32),
            jax.ShapeDtypeStruct((NUM_GRAPHS, NODE_SIZE), jnp.float32),
        ),
        scratch_shapes=[
            pltpu.VMEM((2, HBLK, NODE_SIZE), jnp.float32),
            pltpu.SemaphoreType.DMA,
            pltpu.SemaphoreType.DMA,
        ],
    )(batch.reshape(1, N_NODES), x)
    out_t = pl.pallas_call(
        _tc_mlp,
        out_shape=jax.ShapeDtypeStruct((W2.shape[1], NUM_GRAPHS), jnp.float32),
    )(acc, tsum, cnt, u.T, W1.T, b1.reshape(1, -1), W2, b2.reshape(-1, 1))
    return out_t.T


# speedup vs baseline: 1.0853x; 1.0072x over previous
import functools

import jax
import jax.numpy as jnp
from jax import lax
from jax.experimental import pallas as pl
from jax.experimental.pallas import tpu as pltpu
from jax.experimental.pallas import tpu_sc as plsc

N_NODES = 10000
NODE_SIZE = 128
NUM_GRAPHS = 128
GLOBAL_SIZE = 64
NC = 2
NS = 16
L = 16
C = 224
SC_ROWS = NC * NS * C
SEG_ROWS = 136
ZROWS = 16
CHUNKS = (128, 96)
HBLK = 512

_SELU_SCALE = 1.0507009873554805
_SELU_ALPHA = 1.6732632423543772


def _sc_segsum(x, batch):
    mesh = plsc.VectorSubcoreMesh(core_axis_name="c", subcore_axis_name="s")

    @functools.partial(
        pl.kernel,
        mesh=mesh,
        out_type=jax.ShapeDtypeStruct((NC * SEG_ROWS, NODE_SIZE), jnp.float32),
        scratch_types=[
            pltpu.VMEM((C, NODE_SIZE), jnp.float32),
            pltpu.VMEM((C,), jnp.int32),
            pltpu.VMEM((len(CHUNKS), 128), jnp.int32),
            pltpu.VMEM((ZROWS, NODE_SIZE), jnp.float32),
            pltpu.VMEM_SHARED((SEG_ROWS, NODE_SIZE), jnp.float32),
            pltpu.SemaphoreType.DMA,
            pltpu.SemaphoreType.DMA,
            pltpu.SemaphoreType.DMA,
        ],
    )
    def seg_kernel(x_hbm, b_hbm, acc_out, xv, bv, idx2, zbuf, acc_sh,
                   semx, semb, sems):
        cid = lax.axis_index("c")
        sid = lax.axis_index("s")
        wid = sid * NC + cid
        base = wid * C

        hx = pltpu.async_copy(x_hbm.at[pl.ds(base, C), :], xv, semx)
        hb = pltpu.async_copy(b_hbm.at[pl.ds(base, C)], bv, semb)

        zero = jnp.zeros((L,), jnp.float32)
        for i in range(ZROWS):
            for j in range(NODE_SIZE // L):
                zbuf[i, pl.ds(j * L, L)] = zero
        zbase = jnp.minimum(sid * ZROWS, SEG_ROWS - ZROWS)
        pltpu.sync_copy(zbuf, acc_sh.at[pl.ds(zbase, ZROWS)])

        hb.wait()
        bounds = []
        off = 0
        for csz in CHUNKS:
            bounds.append(off)
            off += csz
        for k in range(C // L):
            p = k * L
            ci = sum(1 for lo in bounds if lo <= p) - 1
            idx2[ci, pl.ds(p - bounds[ci], L)] = bv[pl.ds(p, L)]

        hx.wait()
        plsc.subcore_barrier()

        handles = []
        off = 0
        for ci, csz in enumerate(CHUNKS):
            handles.append(pltpu.async_copy(
                xv.at[pl.ds(off, csz), :],
                acc_sh.at[idx2.at[ci, pl.ds(0, csz)]], sems, add=True))
            off += csz
        for h in handles:
            h.wait()

        plsc.subcore_barrier()

        @pl.when(sid == 0)
        def _():
            pltpu.sync_copy(acc_sh, acc_out.at[pl.ds(cid * SEG_ROWS, SEG_ROWS)])

    return seg_kernel(x, batch)


def _tc_hist_tail(b_ref, x_ref, cnt_ref, tsum_ref, xbuf, sem0, sem1):
    sems = (sem0, sem1)
    blocks = []
    for k in range(0, N_NODES, HBLK):
        blocks.append((k, min(HBLK, N_NODES - k)))
    tail = [(o, s) for (o, s) in blocks if o >= SC_ROWS]

    handles = {}
    for i in range(min(2, len(tail))):
        o, s = tail[i]
        handles[i] = pltpu.make_async_copy(
            x_ref.at[pl.ds(o, s), :], xbuf.at[i % 2, pl.ds(0, s), :],
            sems[i % 2])
        handles[i].start()

    gid = lax.broadcasted_iota(jnp.int32, (NUM_GRAPHS, 1), 0)
    cnt = jnp.zeros((NUM_GRAPHS, 1), jnp.float32)
    tsum = jnp.zeros((NUM_GRAPHS, NODE_SIZE), jnp.float32)
    ti = 0
    for (o, s) in blocks:
        eq = (b_ref[:, o:o + s] == gid).astype(jnp.float32)
        ones_col = jnp.ones((s, 1), jnp.float32)
        cnt = cnt + jnp.dot(eq, ones_col,
                            preferred_element_type=jnp.float32)
        if o >= SC_ROWS:
            handles[ti].wait()
            if ti + 2 < len(tail):
                o2, s2 = tail[ti + 2]
                handles[ti + 2] = pltpu.make_async_copy(
                    x_ref.at[pl.ds(o2, s2), :],
                    xbuf.at[ti % 2, pl.ds(0, s2), :], sems[ti % 2])
                handles[ti + 2].start()
            xblk = xbuf[ti % 2, pl.ds(0, s), :]
            tsum = tsum + jnp.dot(eq, xblk,
                                  preferred_element_type=jnp.float32)
            ti += 1
    cnt_ref[:] = jnp.maximum(cnt, 1.0)
    tsum_ref[:] = tsum


def _tc_mlp(acc_ref, ts_ref, cnt_ref, ut_ref, w1t_ref, b1_ref, w2_ref,
            b2_ref, out_ref):
    s = (acc_ref[:NUM_GRAPHS, :]
         + acc_ref[SEG_ROWS:SEG_ROWS + NUM_GRAPHS, :]
         + ts_ref[:])
    mean = s / cnt_ref[:]
    h = (lax.dot_general(ut_ref[:], w1t_ref[:, :GLOBAL_SIZE],
                         (((0,), (1,)), ((), ())),
                         preferred_element_type=jnp.float32)
         + lax.dot_general(mean, w1t_ref[:, GLOBAL_SIZE:],
                           (((1,), (1,)), ((), ())),
                           preferred_element_type=jnp.float32)
         + b1_ref[:])
    h = _SELU_SCALE * jnp.where(h > 0, h, _SELU_ALPHA * (jnp.exp(h) - 1.0))
    out_t = lax.dot_general(w2_ref[:], h, (((0,), (1,)), ((), ())),
                            preferred_element_type=jnp.float32)
    out_ref[:] = out_t + b2_ref[:]


def kernel(x, edge_index, edge_attr, u, batch, W1, b1, W2, b2):
    acc = _sc_segsum(x, batch)
    cnt, tsum = pl.pallas_call(
        _tc_hist_tail,
        in_specs=[
            pl.BlockSpec(memory_space=pltpu.VMEM),
            pl.BlockSpec(memory_space=pl.ANY),
        ],
        out_shape=(
            jax.ShapeDtypeStruct((NUM_GRAPHS, 1), jnp.float32),
            jax.ShapeDtypeStruct((NUM_GRAPHS, NODE_SIZE), jnp.float32),
        ),
        scratch_shapes=[
            pltpu.VMEM((2, HBLK, NODE_SIZE), jnp.float32),
            pltpu.SemaphoreType.DMA,
            pltpu.SemaphoreType.DMA,
        ],
    )(batch.reshape(1, N_NODES), x)
    out_t = pl.pallas_call(
        _tc_mlp,
        out_shape=jax.ShapeDtypeStruct((W2.shape[1], NUM_GRAPHS), jnp.float32),
    )(acc, tsum, cnt, u.T, W1.T, b1.reshape(1, -1), W2, b2.reshape(-1, 1))
    return out_t.T
